# baseline (speedup 1.0000x reference)
import functools

import jax
import jax.numpy as jnp
from jax import lax
from jax.experimental import pallas as pl
from jax.experimental.pallas import tpu as pltpu
from jax.experimental.pallas import tpu_sc as plsc

_L16 = 16


def _iota16():
    return lax.broadcasted_iota(jnp.int32, (_L16,), 0)


def _pack_body(tT_hbm, tail_hbm, packed_hbm, buf, obuf, gsem0, gsem1, wsem0,
               wsem1, *, num_lines, tail_lines):
    nc = 2
    wid = lax.axis_index("s") * nc + lax.axis_index("c")
    start = 244 * wid + 2 * jnp.minimum(wid, 2)
    n = 244 + 2 * (wid < 2).astype(jnp.int32)
    gsems = (gsem0, gsem1)
    wsems = (wsem0, wsem1)

    def fire(g, b):
        bl = start + g
        pltpu.async_copy(
            tT_hbm.at[:, pl.ds(bl * 128, 128)], buf.at[b], gsems[b]
        )

    def wait_g(g, b):
        bl = start + g
        pltpu.make_async_copy(
            tT_hbm.at[:, pl.ds(bl * 128, 128)], buf.at[b], gsems[b]
        ).wait()

    def transpose(b):
        diags = [lax.bitwise_and(_iota16() + s, 15) for s in range(16)]

        def tvb(t, carry):
            v_vec = 16 * t + _iota16()
            l_vec = lax.shift_right_logical(v_vec, 1)
            colbase = lax.bitwise_and(v_vec, 1) * 64
            for D in range(0, 64, 16):
                d_vecs = [diags[s] + D for s in range(16)]
                vals = [
                    plsc.load_gather(buf.at[b], [d_vecs[s], v_vec])
                    for s in range(16)
                ]
                for s in range(16):
                    plsc.store_scatter(
                        obuf.at[b], [l_vec, colbase + d_vecs[s]], vals[s]
                    )
            return carry

        lax.fori_loop(0, 8, tvb, 0)

    def fire_wb(g, b):
        bl = start + g
        pltpu.async_copy(
            obuf.at[b], packed_hbm.at[pl.ds(bl * 64, 64), :], wsems[b]
        )

    def wait_wb(g, b):
        bl = start + g
        pltpu.make_async_copy(
            obuf.at[b], packed_hbm.at[pl.ds(bl * 64, 64), :], wsems[b]
        ).wait()

    fire(0, 0)
    fire(1, 1)

    def pair(p, carry):
        for b in (0, 1):
            g = 2 * p + b
            wait_g(g, b)

            @pl.when(p >= 1)
            def _():
                wait_wb(g - 2, b)

            transpose(b)
            fire_wb(g, b)

            @pl.when(g + 2 < n)
            def _():
                fire(g + 2, b)

        return carry

    lax.fori_loop(0, n // 2, pair, 0)
    for b in (0, 1):
        wait_wb(n - 2 + b, b)

    @pl.when(wid == 0)
    def _():
        pltpu.sync_copy(tail_hbm, obuf.at[0, pl.ds(0, tail_lines)])
        pltpu.sync_copy(
            obuf.at[0, pl.ds(0, tail_lines)],
            packed_hbm.at[pl.ds(num_lines - tail_lines, tail_lines), :],
        )


def _gather_body(xT_hbm, packed_hbm, outT_hbm, xbuf, gidx, cb, lines, obuf,
                 gsem0, gsem1, wsem0, wsem1, *, seq_len):
    nc = 2
    wid = lax.axis_index("s") * nc + lax.axis_index("c")
    b0 = wid * 128
    gsems = (gsem0, gsem1)
    wsems = (wsem0, wsem1)

    pltpu.sync_copy(xT_hbm.at[:, pl.ds(b0, 128)], xbuf)

    def prep(l, b):
        for c in range(8):
            v = xbuf[l, pl.ds(16 * c, _L16)]
            gidx[b, pl.ds(16 * c, _L16)] = lax.shift_right_logical(v, 1)
            cb[b, pl.ds(16 * c, _L16)] = lax.bitwise_and(v, 1) * 64

    def fire(l, b):
        pltpu.async_copy(packed_hbm.at[gidx.at[b]], lines.at[b], gsems[b])

    def wait_g(b):
        pltpu.make_async_copy(
            packed_hbm.at[gidx.at[b]], lines.at[b], gsems[b]
        ).wait()

    def transpose(b):
        diags = [lax.bitwise_and(_iota16() + s, 15) for s in range(16)]

        def tib(t, carry):
            i_vec = 16 * t + _iota16()
            cbv = cb[b, pl.ds(16 * t, _L16)]
            for D in range(0, 64, 16):
                d_vecs = [diags[s] + D for s in range(16)]
                vals = [
                    plsc.load_gather(lines.at[b], [i_vec, cbv + d_vecs[s]])
                    for s in range(16)
                ]
                for s in range(16):
                    plsc.store_scatter(obuf.at[b], [d_vecs[s], i_vec], vals[s])
            return carry

        lax.fori_loop(0, 8, tib, 0)

    def fire_wb(l, b):
        pltpu.async_copy(
            obuf.at[b], outT_hbm.at[l, :, pl.ds(b0, 128)], wsems[b]
        )

    def wait_wb(l, b):
        pltpu.make_async_copy(
            obuf.at[b], outT_hbm.at[l, :, pl.ds(b0, 128)], wsems[b]
        ).wait()

    for b in (0, 1):
        prep(b, b)
        fire(b, b)

    def pair(p, carry):
        for b in (0, 1):
            l = 2 * p + b
            wait_g(b)

            @pl.when(p >= 1)
            def _():
                wait_wb(l - 2, b)

            transpose(b)
            fire_wb(l, b)

            @pl.when(l + 2 < seq_len)
            def _():
                prep(l + 2, b)
                fire(l + 2, b)

        return carry

    lax.fori_loop(0, seq_len // 2, pair, 0)
    for b in (0, 1):
        wait_wb(seq_len - 2 + b, b)


def kernel(x, table):
    bsz, seq = x.shape
    num_v, d = table.shape
    tT = table.T
    xT = x.T
    num_lines = num_v // 2

    mesh = plsc.VectorSubcoreMesh(core_axis_name="c", subcore_axis_name="s")
    params = pltpu.CompilerParams(
        use_tc_tiling_on_sc=True,
        needs_layout_passes=False,
        disable_bounds_checks=True,
    )

    pack = functools.partial(
        pl.kernel,
        mesh=mesh,
        out_type=jax.ShapeDtypeStruct((num_lines, 128), jnp.float32),
        scratch_types=[
            pltpu.VMEM((2, 64, 128), jnp.float32),
            pltpu.VMEM((2, 64, 128), jnp.float32),
            pltpu.SemaphoreType.DMA,
            pltpu.SemaphoreType.DMA,
            pltpu.SemaphoreType.DMA,
            pltpu.SemaphoreType.DMA,
        ],
        compiler_params=params,
    )(functools.partial(_pack_body, num_lines=num_lines, tail_lines=32))

    gather = functools.partial(
        pl.kernel,
        mesh=mesh,
        out_type=jax.ShapeDtypeStruct((seq, d, bsz), jnp.float32),
        scratch_types=[
            pltpu.VMEM((seq, 128), jnp.int32),
            pltpu.VMEM((2, 128), jnp.int32),
            pltpu.VMEM((2, 128), jnp.int32),
            pltpu.VMEM((2, 128, 128), jnp.float32),
            pltpu.VMEM((2, 64, 128), jnp.float32),
            pltpu.SemaphoreType.DMA,
            pltpu.SemaphoreType.DMA,
            pltpu.SemaphoreType.DMA,
            pltpu.SemaphoreType.DMA,
        ],
        compiler_params=params,
    )(functools.partial(_gather_body, seq_len=seq))

    full_v = (num_v // 128) * 128
    tail_packed = table[full_v:].reshape(-1, 128)
    packed = pack(tT, tail_packed)
    outT = gather(xT, packed)
    return jnp.transpose(outT, (2, 0, 1))


# speedup vs baseline: 1.2818x; 1.2818x over previous
import functools

import jax
import jax.numpy as jnp
from jax import lax
from jax.experimental import pallas as pl
from jax.experimental.pallas import tpu as pltpu
from jax.experimental.pallas import tpu_sc as plsc

_L16 = 16


def _iota16():
    return lax.broadcasted_iota(jnp.int32, (_L16,), 0)


def _pack_body(tT_hbm, tail_hbm, packed_hbm, buf, obuf, gsem0, gsem1, wsem0,
               wsem1, *, num_lines, tail_lines):
    nc = 2
    wid = lax.axis_index("s") * nc + lax.axis_index("c")
    start = 244 * wid + 2 * jnp.minimum(wid, 2)
    n = 244 + 2 * (wid < 2).astype(jnp.int32)
    gsems = (gsem0, gsem1)
    wsems = (wsem0, wsem1)

    def fire(g, b):
        bl = start + g
        pltpu.async_copy(
            tT_hbm.at[:, pl.ds(bl * 128, 128)], buf.at[b], gsems[b]
        )

    def wait_g(g, b):
        bl = start + g
        pltpu.make_async_copy(
            tT_hbm.at[:, pl.ds(bl * 128, 128)], buf.at[b], gsems[b]
        ).wait()

    iota = _iota16()
    v_vecs = [16 * t + iota for t in range(8)]
    l_vecs = [8 * t + lax.shift_right_logical(iota, 1) for t in range(8)]
    colbase = lax.bitwise_and(iota, 1) * 64

    def transpose(b):
        def dsf(k, carry):
            d_vec = lax.bitwise_and(iota + k, 15) + lax.bitwise_and(k, 48)
            col_vec = colbase + d_vec
            vals = [
                plsc.load_gather(buf.at[b], [d_vec, v_vecs[t]])
                for t in range(8)
            ]
            for t in range(8):
                plsc.store_scatter(obuf.at[b], [l_vecs[t], col_vec], vals[t])
            return carry

        lax.fori_loop(0, 64, dsf, 0)

    def fire_wb(g, b):
        bl = start + g
        pltpu.async_copy(
            obuf.at[b], packed_hbm.at[pl.ds(bl * 64, 64), :], wsems[b]
        )

    def wait_wb(g, b):
        bl = start + g
        pltpu.make_async_copy(
            obuf.at[b], packed_hbm.at[pl.ds(bl * 64, 64), :], wsems[b]
        ).wait()

    fire(0, 0)
    fire(1, 1)

    def pair(p, carry):
        for b in (0, 1):
            g = 2 * p + b
            wait_g(g, b)

            @pl.when(p >= 1)
            def _():
                wait_wb(g - 2, b)

            transpose(b)
            fire_wb(g, b)

            @pl.when(g + 2 < n)
            def _():
                fire(g + 2, b)

        return carry

    lax.fori_loop(0, n // 2, pair, 0)
    for b in (0, 1):
        wait_wb(n - 2 + b, b)

    @pl.when(wid == 0)
    def _():
        pltpu.sync_copy(tail_hbm, obuf.at[0, pl.ds(0, tail_lines)])
        pltpu.sync_copy(
            obuf.at[0, pl.ds(0, tail_lines)],
            packed_hbm.at[pl.ds(num_lines - tail_lines, tail_lines), :],
        )


def _gather_body(xT_hbm, packed_hbm, outT_hbm, xbuf, gidx, cb, lines, obuf,
                 gsem0, gsem1, wsem0, wsem1, *, seq_len):
    nc = 2
    wid = lax.axis_index("s") * nc + lax.axis_index("c")
    b0 = wid * 128
    gsems = (gsem0, gsem1)
    wsems = (wsem0, wsem1)

    pltpu.sync_copy(xT_hbm.at[:, pl.ds(b0, 128)], xbuf)

    def prep(l, b):
        for c in range(8):
            v = xbuf[l, pl.ds(16 * c, _L16)]
            gidx[b, pl.ds(16 * c, _L16)] = lax.shift_right_logical(v, 1)
            cb[b, pl.ds(16 * c, _L16)] = lax.bitwise_and(v, 1) * 64

    def fire(l, b):
        pltpu.async_copy(packed_hbm.at[gidx.at[b]], lines.at[b], gsems[b])

    def wait_g(b):
        pltpu.make_async_copy(
            packed_hbm.at[gidx.at[b]], lines.at[b], gsems[b]
        ).wait()

    iota = _iota16()
    i_vecs = [16 * t + iota for t in range(8)]

    def transpose(b):
        cbvs = [cb[b, pl.ds(16 * t, _L16)] for t in range(8)]

        def dsf(k, carry):
            d_vec = lax.bitwise_and(iota + k, 15) + lax.bitwise_and(k, 48)
            vals = [
                plsc.load_gather(lines.at[b], [i_vecs[t], cbvs[t] + d_vec])
                for t in range(8)
            ]
            for t in range(8):
                plsc.store_scatter(obuf.at[b], [d_vec, i_vecs[t]], vals[t])
            return carry

        lax.fori_loop(0, 64, dsf, 0)

    def fire_wb(l, b):
        pltpu.async_copy(
            obuf.at[b], outT_hbm.at[l, :, pl.ds(b0, 128)], wsems[b]
        )

    def wait_wb(l, b):
        pltpu.make_async_copy(
            obuf.at[b], outT_hbm.at[l, :, pl.ds(b0, 128)], wsems[b]
        ).wait()

    for b in (0, 1):
        prep(b, b)
        fire(b, b)

    def pair(p, carry):
        for b in (0, 1):
            l = 2 * p + b
            wait_g(b)

            @pl.when(p >= 1)
            def _():
                wait_wb(l - 2, b)

            transpose(b)
            fire_wb(l, b)

            @pl.when(l + 2 < seq_len)
            def _():
                prep(l + 2, b)
                fire(l + 2, b)

        return carry

    lax.fori_loop(0, seq_len // 2, pair, 0)
    for b in (0, 1):
        wait_wb(seq_len - 2 + b, b)


def kernel(x, table):
    bsz, seq = x.shape
    num_v, d = table.shape
    tT = table.T
    xT = x.T
    num_lines = num_v // 2

    mesh = plsc.VectorSubcoreMesh(core_axis_name="c", subcore_axis_name="s")
    params = pltpu.CompilerParams(
        use_tc_tiling_on_sc=True,
        needs_layout_passes=False,
        disable_bounds_checks=True,
    )

    pack = functools.partial(
        pl.kernel,
        mesh=mesh,
        out_type=jax.ShapeDtypeStruct((num_lines, 128), jnp.float32),
        scratch_types=[
            pltpu.VMEM((2, 64, 128), jnp.float32),
            pltpu.VMEM((2, 64, 128), jnp.float32),
            pltpu.SemaphoreType.DMA,
            pltpu.SemaphoreType.DMA,
            pltpu.SemaphoreType.DMA,
            pltpu.SemaphoreType.DMA,
        ],
        compiler_params=params,
    )(functools.partial(_pack_body, num_lines=num_lines, tail_lines=32))

    gather = functools.partial(
        pl.kernel,
        mesh=mesh,
        out_type=jax.ShapeDtypeStruct((seq, d, bsz), jnp.float32),
        scratch_types=[
            pltpu.VMEM((seq, 128), jnp.int32),
            pltpu.VMEM((2, 128), jnp.int32),
            pltpu.VMEM((2, 128), jnp.int32),
            pltpu.VMEM((2, 128, 128), jnp.float32),
            pltpu.VMEM((2, 64, 128), jnp.float32),
            pltpu.SemaphoreType.DMA,
            pltpu.SemaphoreType.DMA,
            pltpu.SemaphoreType.DMA,
            pltpu.SemaphoreType.DMA,
        ],
        compiler_params=params,
    )(functools.partial(_gather_body, seq_len=seq))

    full_v = (num_v // 128) * 128
    tail_packed = table[full_v:].reshape(-1, 128)
    packed = pack(tT, tail_packed)
    outT = gather(xT, packed)
    return jnp.transpose(outT, (2, 0, 1))


# speedup vs baseline: 1.2827x; 1.0007x over previous
import functools

import jax
import jax.numpy as jnp
from jax import lax
from jax.experimental import pallas as pl
from jax.experimental.pallas import tpu as pltpu
from jax.experimental.pallas import tpu_sc as plsc

_L16 = 16


def _iota16():
    return lax.broadcasted_iota(jnp.int32, (_L16,), 0)


def _pack_body(tT_hbm, tail_hbm, packed_hbm, buf, obuf, gsem0, gsem1, wsem0,
               wsem1, *, num_lines, tail_lines):
    nc = 2
    wid = lax.axis_index("s") * nc + lax.axis_index("c")
    start = 244 * wid + 2 * jnp.minimum(wid, 2)
    n = 244 + 2 * (wid < 2).astype(jnp.int32)
    gsems = (gsem0, gsem1)
    wsems = (wsem0, wsem1)

    def fire(g, b):
        bl = start + g
        pltpu.async_copy(
            tT_hbm.at[:, pl.ds(bl * 128, 128)], buf.at[b], gsems[b]
        )

    def wait_g(g, b):
        bl = start + g
        pltpu.make_async_copy(
            tT_hbm.at[:, pl.ds(bl * 128, 128)], buf.at[b], gsems[b]
        ).wait()

    iota = _iota16()
    v_vecs = [16 * t + iota for t in range(8)]
    l_vecs = [8 * t + lax.shift_right_logical(iota, 1) for t in range(8)]
    colbase = lax.bitwise_and(iota, 1) * 64

    def transpose(b):
        def dsf(k2, carry):
            for u in (0, 1):
                k = 2 * k2 + u
                d_vec = lax.bitwise_and(iota + k, 15) + lax.bitwise_and(
                    k, 48
                )
                col_vec = colbase + d_vec
                vals = [
                    plsc.load_gather(buf.at[b], [d_vec, v_vecs[t]])
                    for t in range(8)
                ]
                for t in range(8):
                    plsc.store_scatter(
                        obuf.at[b], [l_vecs[t], col_vec], vals[t]
                    )
            return carry

        lax.fori_loop(0, 32, dsf, 0)

    def fire_wb(g, b):
        bl = start + g
        pltpu.async_copy(
            obuf.at[b], packed_hbm.at[pl.ds(bl * 64, 64), :], wsems[b]
        )

    def wait_wb(g, b):
        bl = start + g
        pltpu.make_async_copy(
            obuf.at[b], packed_hbm.at[pl.ds(bl * 64, 64), :], wsems[b]
        ).wait()

    fire(0, 0)
    fire(1, 1)

    def pair(p, carry):
        for b in (0, 1):
            g = 2 * p + b
            wait_g(g, b)

            @pl.when(p >= 1)
            def _():
                wait_wb(g - 2, b)

            transpose(b)
            fire_wb(g, b)

            @pl.when(g + 2 < n)
            def _():
                fire(g + 2, b)

        return carry

    lax.fori_loop(0, n // 2, pair, 0)
    for b in (0, 1):
        wait_wb(n - 2 + b, b)

    @pl.when(wid == 0)
    def _():
        pltpu.sync_copy(tail_hbm, obuf.at[0, pl.ds(0, tail_lines)])
        pltpu.sync_copy(
            obuf.at[0, pl.ds(0, tail_lines)],
            packed_hbm.at[pl.ds(num_lines - tail_lines, tail_lines), :],
        )


def _gather_body(xT_hbm, packed_hbm, outT_hbm, xbuf, gidx, cb, lines, obuf,
                 gsem0, gsem1, wsem0, wsem1, *, seq_len):
    nc = 2
    wid = lax.axis_index("s") * nc + lax.axis_index("c")
    b0 = wid * 128
    gsems = (gsem0, gsem1)
    wsems = (wsem0, wsem1)

    pltpu.sync_copy(xT_hbm.at[:, pl.ds(b0, 128)], xbuf)

    def prep(l, b):
        for c in range(8):
            v = xbuf[l, pl.ds(16 * c, _L16)]
            gidx[b, pl.ds(16 * c, _L16)] = lax.shift_right_logical(v, 1)
            cb[b, pl.ds(16 * c, _L16)] = lax.bitwise_and(v, 1) * 64

    def fire(l, b):
        pltpu.async_copy(packed_hbm.at[gidx.at[b]], lines.at[b], gsems[b])

    def wait_g(b):
        pltpu.make_async_copy(
            packed_hbm.at[gidx.at[b]], lines.at[b], gsems[b]
        ).wait()

    iota = _iota16()
    i_vecs = [16 * t + iota for t in range(8)]

    def transpose(b):
        cbvs = [cb[b, pl.ds(16 * t, _L16)] for t in range(8)]

        def dsf(k2, carry):
            for u in (0, 1):
                k = 2 * k2 + u
                d_vec = lax.bitwise_and(iota + k, 15) + lax.bitwise_and(
                    k, 48
                )
                cols = [cbvs[t] + d_vec for t in range(8)]
                vals = [
                    plsc.load_gather(lines.at[b], [i_vecs[t], cols[t]])
                    for t in range(8)
                ]
                for t in range(8):
                    plsc.store_scatter(
                        obuf.at[b], [d_vec, i_vecs[t]], vals[t]
                    )
            return carry

        lax.fori_loop(0, 32, dsf, 0)

    def fire_wb(l, b):
        pltpu.async_copy(
            obuf.at[b], outT_hbm.at[l, :, pl.ds(b0, 128)], wsems[b]
        )

    def wait_wb(l, b):
        pltpu.make_async_copy(
            obuf.at[b], outT_hbm.at[l, :, pl.ds(b0, 128)], wsems[b]
        ).wait()

    for b in (0, 1):
        prep(b, b)
        fire(b, b)

    def pair(p, carry):
        for b in (0, 1):
            l = 2 * p + b
            wait_g(b)

            @pl.when(p >= 1)
            def _():
                wait_wb(l - 2, b)

            transpose(b)
            fire_wb(l, b)

            @pl.when(l + 2 < seq_len)
            def _():
                prep(l + 2, b)
                fire(l + 2, b)

        return carry

    lax.fori_loop(0, seq_len // 2, pair, 0)
    for b in (0, 1):
        wait_wb(seq_len - 2 + b, b)


def kernel(x, table):
    bsz, seq = x.shape
    num_v, d = table.shape
    tT = table.T
    xT = x.T
    num_lines = num_v // 2

    mesh = plsc.VectorSubcoreMesh(core_axis_name="c", subcore_axis_name="s")
    params = pltpu.CompilerParams(
        use_tc_tiling_on_sc=True,
        needs_layout_passes=False,
        disable_bounds_checks=True,
    )

    pack = functools.partial(
        pl.kernel,
        mesh=mesh,
        out_type=jax.ShapeDtypeStruct((num_lines, 128), jnp.float32),
        scratch_types=[
            pltpu.VMEM((2, 64, 128), jnp.float32),
            pltpu.VMEM((2, 64, 128), jnp.float32),
            pltpu.SemaphoreType.DMA,
            pltpu.SemaphoreType.DMA,
            pltpu.SemaphoreType.DMA,
            pltpu.SemaphoreType.DMA,
        ],
        compiler_params=params,
    )(functools.partial(_pack_body, num_lines=num_lines, tail_lines=32))

    gather = functools.partial(
        pl.kernel,
        mesh=mesh,
        out_type=jax.ShapeDtypeStruct((seq, d, bsz), jnp.float32),
        scratch_types=[
            pltpu.VMEM((seq, 128), jnp.int32),
            pltpu.VMEM((2, 128), jnp.int32),
            pltpu.VMEM((2, 128), jnp.int32),
            pltpu.VMEM((2, 128, 128), jnp.float32),
            pltpu.VMEM((2, 64, 128), jnp.float32),
            pltpu.SemaphoreType.DMA,
            pltpu.SemaphoreType.DMA,
            pltpu.SemaphoreType.DMA,
            pltpu.SemaphoreType.DMA,
        ],
        compiler_params=params,
    )(functools.partial(_gather_body, seq_len=seq))

    full_v = (num_v // 128) * 128
    tail_packed = table[full_v:].reshape(-1, 128)
    packed = pack(tT, tail_packed)
    outT = gather(xT, packed)
    return jnp.transpose(outT, (2, 0, 1))
